# Initial kernel scaffold; baseline (speedup 1.0000x reference)
import jax
import jax.numpy as jnp
from jax.experimental import pallas as pl


def kernel(pos, pos_embedding):
    raise NotImplementedError("write your pallas kernel here")


# speedup vs baseline: 2.3858x; 2.3858x over previous
import functools

import jax
import jax.numpy as jnp
from jax import lax
from jax.experimental import pallas as pl
from jax.experimental.pallas import tpu as pltpu
from jax.experimental.pallas import tpu_sc as plsc

DIMS = 1024
NUM_IDX = 4 * 8192
NC, NS = 2, 16
NW = NC * NS
B_PER_W = NUM_IDX // NW
CHUNK = 32
N_CHUNKS = B_PER_W // CHUNK


@functools.partial(
    pl.kernel,
    out_type=jax.ShapeDtypeStruct((NUM_IDX, DIMS), jnp.float32),
    mesh=plsc.VectorSubcoreMesh(
        core_axis_name="c", subcore_axis_name="s", num_cores=NC,
        num_subcores=NS),
    scratch_types=[
        pltpu.VMEM((B_PER_W,), jnp.int32),
        pltpu.VMEM((2, CHUNK, DIMS), jnp.float32),
        pltpu.SemaphoreType.DMA,
        pltpu.SemaphoreType.DMA,
        pltpu.SemaphoreType.DMA,
    ],
)
def _sc_gather(idx_hbm, table_hbm, out_hbm, idx_v, rows_v, gsem0, gsem1,
               ssem):
    wid = lax.axis_index("s") * NC + lax.axis_index("c")
    base = wid * B_PER_W
    pltpu.sync_copy(idx_hbm.at[pl.ds(base, B_PER_W)], idx_v)

    gsems = (gsem0, gsem1)

    def start_gather(g, buf):
        return pltpu.async_copy(
            table_hbm.at[idx_v.at[pl.ds(g * CHUNK, CHUNK)]],
            rows_v.at[buf], gsems[buf])

    start_gather(0, 0)
    start_gather(1, 1)

    def body(g, _):
        buf = lax.rem(g, 2)

        @pl.when(buf == 0)
        def _():
            pltpu.make_async_copy(
                table_hbm.at[idx_v.at[pl.ds(0, CHUNK)]],
                rows_v.at[0], gsem0).wait()
            pltpu.async_copy(rows_v.at[0],
                             out_hbm.at[pl.ds(base + g * CHUNK, CHUNK)],
                             ssem).wait()

            @pl.when(g + 2 < N_CHUNKS)
            def _():
                start_gather(g + 2, 0)

        @pl.when(buf == 1)
        def _():
            pltpu.make_async_copy(
                table_hbm.at[idx_v.at[pl.ds(0, CHUNK)]],
                rows_v.at[1], gsem1).wait()
            pltpu.async_copy(rows_v.at[1],
                             out_hbm.at[pl.ds(base + g * CHUNK, CHUNK)],
                             ssem).wait()

            @pl.when(g + 2 < N_CHUNKS)
            def _():
                start_gather(g + 2, 1)

        return 0

    lax.fori_loop(0, N_CHUNKS, body, 0)


def kernel(pos, pos_embedding):
    idx = pos.reshape(-1).astype(jnp.int32)
    out = _sc_gather(idx, pos_embedding)
    return out.reshape(pos.shape + (DIMS,))
